# baseline (device time: 176309 ns/iter reference)
import jax
import jax.numpy as jnp
from jax import lax
from jax.experimental import pallas as pl
from jax.experimental.pallas import tpu as pltpu

N_DEV = 16
B, S, D = 2, 256, 1024
DC, H, DH, DR = 64, 16, 64, 32
ROWS = 2 * B * S
CHUNK = ROWS // N_DEV
SCALE = (DH + DR) ** -0.5


def kernel(x, Wdkv, Wuk, Wuv, Wq, Wqr, Wkr, Wo):
    def body(x_ref, wdkv_ref, wuk_ref, wuv_ref, wq_ref, wqr_ref, wkr_ref,
             wo_ref, out_ref, acc_ref, rs_buf, q_ref, qr_ref, kr_ref, o_ref,
             send_sem, rs_sems, ag_sems):
        my = lax.axis_index("i")
        left = lax.rem(my + N_DEV - 1, N_DEV)
        right = lax.rem(my + 1, N_DEV)

        barrier = pltpu.get_barrier_semaphore()
        for nbr in (left, right):
            pl.semaphore_signal(barrier, inc=1, device_id=(nbr,),
                                device_id_type=pl.DeviceIdType.MESH)
        pl.semaphore_wait(barrier, 2)

        for b in range(B):
            xb = x_ref[b]
            c = jnp.dot(xb, wdkv_ref[...],
                        preferred_element_type=jnp.float32)
            acc_ref[b * S:(b + 1) * S, :] = jnp.dot(
                c, wuk_ref[...], preferred_element_type=jnp.float32)
            acc_ref[(B + b) * S:(B + b + 1) * S, :] = jnp.dot(
                c, wuv_ref[...], preferred_element_type=jnp.float32)
            q_ref[b * S:(b + 1) * S, :] = jnp.dot(
                xb, wq_ref[...], preferred_element_type=jnp.float32)
            qr_ref[b * S:(b + 1) * S, :] = jnp.dot(
                xb, wqr_ref[...], preferred_element_type=jnp.float32)
            kr_ref[b * S:(b + 1) * S, :] = jnp.dot(
                xb, wkr_ref[...], preferred_element_type=jnp.float32)

        for t in range(N_DEV - 1):
            sc = lax.rem(my + 2 * N_DEV - t, N_DEV)
            rdma = pltpu.make_async_remote_copy(
                src_ref=acc_ref.at[pl.ds(sc * CHUNK, CHUNK), :],
                dst_ref=rs_buf.at[t],
                send_sem=send_sem,
                recv_sem=rs_sems.at[t],
                device_id=(right,),
                device_id_type=pl.DeviceIdType.MESH,
            )
            rdma.start()
            rdma.wait()
            rc = lax.rem(my + 2 * N_DEV - t - 1, N_DEV)
            acc_ref[pl.ds(rc * CHUNK, CHUNK), :] = (
                acc_ref[pl.ds(rc * CHUNK, CHUNK), :] + rs_buf[t])

        for t in range(N_DEV - 1):
            sc = lax.rem(my + 1 + 2 * N_DEV - t, N_DEV)
            rdma = pltpu.make_async_remote_copy(
                src_ref=acc_ref.at[pl.ds(sc * CHUNK, CHUNK), :],
                dst_ref=acc_ref.at[pl.ds(sc * CHUNK, CHUNK), :],
                send_sem=send_sem,
                recv_sem=ag_sems.at[t],
                device_id=(right,),
                device_id_type=pl.DeviceIdType.MESH,
            )
            rdma.start()
            rdma.wait()

        for b in range(B):
            qr_b = qr_ref[b * S:(b + 1) * S, :]
            kr_b = kr_ref[b * S:(b + 1) * S, :]
            for h in range(H):
                q = q_ref[b * S:(b + 1) * S, h * DH:(h + 1) * DH]
                k = acc_ref[b * S:(b + 1) * S, h * DH:(h + 1) * DH]
                v = acc_ref[(B + b) * S:(B + b + 1) * S, h * DH:(h + 1) * DH]
                qr = qr_b[:, h * DR:(h + 1) * DR]
                s = lax.dot_general(q, k, (((1,), (1,)), ((), ())),
                                    preferred_element_type=jnp.float32)
                s = s + lax.dot_general(qr, kr_b, (((1,), (1,)), ((), ())),
                                        preferred_element_type=jnp.float32)
                s = s * SCALE
                m = jnp.max(s, axis=1, keepdims=True)
                p = jnp.exp(s - m)
                p = p / jnp.sum(p, axis=1, keepdims=True)
                o_ref[b * S:(b + 1) * S, h * DH:(h + 1) * DH] = jnp.dot(
                    p, v, preferred_element_type=jnp.float32)

        for b in range(B):
            out_ref[b] = jnp.dot(o_ref[b * S:(b + 1) * S, :], wo_ref[...],
                                 preferred_element_type=jnp.float32)

    return pl.pallas_call(
        body,
        out_shape=jax.ShapeDtypeStruct((B, S, D), jnp.float32),
        in_specs=[pl.BlockSpec(memory_space=pltpu.VMEM)] * 8,
        out_specs=pl.BlockSpec(memory_space=pltpu.VMEM),
        scratch_shapes=[
            pltpu.VMEM((ROWS, D), jnp.float32),
            pltpu.VMEM((N_DEV - 1, CHUNK, D), jnp.float32),
            pltpu.VMEM((B * S, D), jnp.float32),
            pltpu.VMEM((B * S, H * DR), jnp.float32),
            pltpu.VMEM((B * S, DR), jnp.float32),
            pltpu.VMEM((B * S, D), jnp.float32),
            pltpu.SemaphoreType.DMA,
            pltpu.SemaphoreType.DMA((N_DEV - 1,)),
            pltpu.SemaphoreType.DMA((N_DEV - 1,)),
        ],
        compiler_params=pltpu.CompilerParams(collective_id=0),
    )(x, Wdkv, Wuk, Wuv, Wq, Wqr, Wkr, Wo)


# device time: 100706 ns/iter; 1.7507x vs baseline; 1.7507x over previous
import jax
import jax.numpy as jnp
from jax import lax
from jax.experimental import pallas as pl
from jax.experimental.pallas import tpu as pltpu

N_DEV = 16
B, S, D = 2, 256, 1024
DC, H, DH, DR = 64, 16, 64, 32
ROWS = 2 * B * S
CH = (ROWS // 2) // N_DEV
SCALE = (DH + DR) ** -0.5

BITS = [8, 4, 2, 1]
SCR_OFF = [0, 256, 384, 448]


def _logical(x, y, zlo, zhi):
    w = 2 * y + (x + y - 2 * x * y)
    return 4 * (zlo + 2 * zhi) + w


def kernel(x, Wdkv, Wuk, Wuv, Wq, Wqr, Wkr, Wo):
    def body(x_ref, wdkv_ref, wuk_ref, wuv_ref, wq_ref, wqr_ref, wkr_ref,
             wo_ref, out_ref, acc_ref, scr_ref, q_ref, qr_ref, kr_ref, o_ref,
             send_semA, send_semB, rsA, agA, rsB, agB):
        my = lax.axis_index("i")
        w = lax.rem(my, 4)
        z = my // 4
        cx = jnp.logical_or(w == 1, w == 2).astype(jnp.int32)
        cy = (w >= 2).astype(jnp.int32)
        zlo = lax.rem(z, 2)
        zhi = z // 2

        eA = 8 * cx + 4 * zlo + 2 * cy + zhi
        eB = 8 * cy + 4 * cx + 2 * zhi + zlo
        p_x = _logical(1 - cx, cy, zlo, zhi)
        p_y = _logical(cx, 1 - cy, zlo, zhi)
        p_zlo = _logical(cx, cy, 1 - zlo, zhi)
        p_zhi = _logical(cx, cy, zlo, 1 - zhi)
        partsA = [p_x, p_zlo, p_y, p_zhi]
        partsB = [p_y, p_x, p_zhi, p_zlo]

        barrier = pltpu.get_barrier_semaphore()
        for nbr in (p_x, p_y, p_zlo, p_zhi):
            pl.semaphore_signal(barrier, inc=1, device_id=(nbr,),
                                device_id_type=pl.DeviceIdType.MESH)
        pl.semaphore_wait(barrier, 4)

        for b in range(B):
            xb = x_ref[b]
            c = jnp.dot(xb, wdkv_ref[...],
                        preferred_element_type=jnp.float32)
            acc_ref[b * S:(b + 1) * S, :] = jnp.dot(
                c, wuk_ref[...], preferred_element_type=jnp.float32)
            acc_ref[(B + b) * S:(B + b + 1) * S, :] = jnp.dot(
                c, wuv_ref[...], preferred_element_type=jnp.float32)

        paths = [
            dict(e=eA, off=0, scr=0, parts=partsA, ssem=send_semA,
                 rs=rsA, ag=agA),
            dict(e=eB, off=ROWS // 2, scr=480, parts=partsB, ssem=send_semB,
                 rs=rsB, ag=agB),
        ]

        for k in range(4):
            b_ = BITS[k]
            L = 2 * b_
            rdmas = []
            for p in paths:
                e = p["e"]
                start = (e // L) * L
                ebit = lax.rem(e // b_, 2)
                send_c = start + (1 - ebit) * b_
                keep_c = start + ebit * b_
                so = p["scr"] + SCR_OFF[k]
                rdma = pltpu.make_async_remote_copy(
                    src_ref=acc_ref.at[pl.ds(p["off"] + send_c * CH,
                                             b_ * CH), :],
                    dst_ref=scr_ref.at[so:so + b_ * CH, :],
                    send_sem=p["ssem"],
                    recv_sem=p["rs"].at[k],
                    device_id=(p["parts"][k],),
                    device_id_type=pl.DeviceIdType.MESH,
                )
                rdma.start()
                rdmas.append((rdma, p["off"] + keep_c * CH, so))
            if k == 0:
                for b in range(B):
                    xb = x_ref[b]
                    q_ref[b * S:(b + 1) * S, :] = jnp.dot(
                        xb, wq_ref[...], preferred_element_type=jnp.float32)
                    qr_ref[b * S:(b + 1) * S, :] = jnp.dot(
                        xb, wqr_ref[...], preferred_element_type=jnp.float32)
                    kr_ref[b * S:(b + 1) * S, :] = jnp.dot(
                        xb, wkr_ref[...], preferred_element_type=jnp.float32)
            for rdma, keep_row, so in rdmas:
                rdma.wait()
                acc_ref[pl.ds(keep_row, b_ * CH), :] = (
                    acc_ref[pl.ds(keep_row, b_ * CH), :]
                    + scr_ref[so:so + b_ * CH, :])

        for j in range(4):
            L = 1 << j
            rdmas = []
            for p in paths:
                e = p["e"]
                start = (e // L) * L
                rdma = pltpu.make_async_remote_copy(
                    src_ref=acc_ref.at[pl.ds(p["off"] + start * CH,
                                             L * CH), :],
                    dst_ref=acc_ref.at[pl.ds(p["off"] + start * CH,
                                             L * CH), :],
                    send_sem=p["ssem"],
                    recv_sem=p["ag"].at[j],
                    device_id=(p["parts"][3 - j],),
                    device_id_type=pl.DeviceIdType.MESH,
                )
                rdma.start()
                rdmas.append(rdma)
            for rdma in rdmas:
                rdma.wait()

        for b in range(B):
            qr_b = qr_ref[b * S:(b + 1) * S, :]
            kr_b = kr_ref[b * S:(b + 1) * S, :]
            for h in range(H):
                q = q_ref[b * S:(b + 1) * S, h * DH:(h + 1) * DH]
                k_ = acc_ref[b * S:(b + 1) * S, h * DH:(h + 1) * DH]
                v = acc_ref[(B + b) * S:(B + b + 1) * S, h * DH:(h + 1) * DH]
                qr = qr_b[:, h * DR:(h + 1) * DR]
                s = lax.dot_general(q, k_, (((1,), (1,)), ((), ())),
                                    preferred_element_type=jnp.float32)
                s = s + lax.dot_general(qr, kr_b, (((1,), (1,)), ((), ())),
                                        preferred_element_type=jnp.float32)
                s = s * SCALE
                m = jnp.max(s, axis=1, keepdims=True)
                pr = jnp.exp(s - m)
                pr = pr / jnp.sum(pr, axis=1, keepdims=True)
                o_ref[b * S:(b + 1) * S, h * DH:(h + 1) * DH] = jnp.dot(
                    pr, v, preferred_element_type=jnp.float32)

        for b in range(B):
            out_ref[b] = jnp.dot(o_ref[b * S:(b + 1) * S, :], wo_ref[...],
                                 preferred_element_type=jnp.float32)

    return pl.pallas_call(
        body,
        out_shape=jax.ShapeDtypeStruct((B, S, D), jnp.float32),
        in_specs=[pl.BlockSpec(memory_space=pltpu.VMEM)] * 8,
        out_specs=pl.BlockSpec(memory_space=pltpu.VMEM),
        scratch_shapes=[
            pltpu.VMEM((ROWS, D), jnp.float32),
            pltpu.VMEM((960, D), jnp.float32),
            pltpu.VMEM((B * S, D), jnp.float32),
            pltpu.VMEM((B * S, H * DR), jnp.float32),
            pltpu.VMEM((B * S, DR), jnp.float32),
            pltpu.VMEM((B * S, D), jnp.float32),
            pltpu.SemaphoreType.DMA,
            pltpu.SemaphoreType.DMA,
            pltpu.SemaphoreType.DMA((4,)),
            pltpu.SemaphoreType.DMA((4,)),
            pltpu.SemaphoreType.DMA((4,)),
            pltpu.SemaphoreType.DMA((4,)),
        ],
        compiler_params=pltpu.CompilerParams(collective_id=0),
    )(x, Wdkv, Wuk, Wuv, Wq, Wqr, Wkr, Wo)


# device time: 72706 ns/iter; 2.4250x vs baseline; 1.3851x over previous
import jax
import jax.numpy as jnp
from jax import lax
from jax.experimental import pallas as pl
from jax.experimental.pallas import tpu as pltpu

N_DEV = 16
B, S, D = 2, 256, 1024
DC, H, DH, DR = 64, 16, 64, 32
CHT = 2 * DH
CHO = DH
SCALE = (DH + DR) ** -0.5

BITS_A = [8, 4, 2, 1]
SCR_ROWS = [8 * CHT, 4 * CHT, 2 * CHT, 1 * CHT]
SCR_OFF = [0, 1024, 1536, 1792]
DG = lambda a, b_, dims: lax.dot_general(
    a, b_, (dims, ((), ())), preferred_element_type=jnp.float32)


def _logical(x, y, zlo, zhi):
    w = 2 * y + (x + y - 2 * x * y)
    return 4 * (zlo + 2 * zhi) + w


def kernel(x, Wdkv, Wuk, Wuv, Wq, Wqr, Wkr, Wo):
    def body(x_ref, wdkv_ref, wuk_ref, wuv_ref, wq_ref, wqr_ref, wkr_ref,
             wo_ref, out_ref, accT, scr, ctT, qT, qrT, krT, oT,
             sendA, sendB, rsA, rsB, agA, agB):
        my = lax.axis_index("i")
        w = lax.rem(my, 4)
        z = my // 4
        cx = jnp.logical_or(w == 1, w == 2).astype(jnp.int32)
        cy = (w >= 2).astype(jnp.int32)
        zlo = lax.rem(z, 2)
        zhi = z // 2

        e = 8 * cx + 4 * zlo + 2 * cy + zhi
        e1 = lax.rem(e, 2)
        e2 = lax.rem(e // 2, 2)
        e4 = lax.rem(e // 4, 2)
        e8 = e // 8
        p_x = _logical(1 - cx, cy, zlo, zhi)
        p_y = _logical(cx, 1 - cy, zlo, zhi)
        p_zlo = _logical(cx, cy, 1 - zlo, zhi)
        p_zhi = _logical(cx, cy, zlo, 1 - zhi)
        a0 = 8 * e8 + 4 * e4
        a4 = (e // 4) * 4

        barrier = pltpu.get_barrier_semaphore()
        for nbr in (p_x, p_y, p_zlo, p_zhi):
            pl.semaphore_signal(barrier, inc=1, device_id=(nbr,),
                                device_id_type=pl.DeviceIdType.MESH)
        pl.semaphore_wait(barrier, 4)

        for b in range(B):
            ctT[:, b * S:(b + 1) * S] = DG(wdkv_ref[...], x_ref[b],
                                           ((0,), (1,)))
        ct = ctT[...]
        for h in range(H):
            accT[h * CHT:h * CHT + DH, :] = DG(
                wuk_ref[:, h * DH:(h + 1) * DH], ct, ((0,), (0,)))
            accT[h * CHT + DH:(h + 1) * CHT, :] = DG(
                wuv_ref[:, h * DH:(h + 1) * DH], ct, ((0,), (0,)))

        rs_plan = {
            "A": [([(((e // (2 * b_)) * (2 * b_)) + (1 - lax.rem(e // b_, 2)) * b_, b_)],
                   [(((e // (2 * b_)) * (2 * b_)) + lax.rem(e // b_, 2) * b_, b_)],
                   p) for b_, p in zip(BITS_A, [p_x, p_zlo, p_y, p_zhi])],
            "B": [
                ([(4 * (1 - e4), 4), (8 + 4 * (1 - e4), 4)],
                 [(4 * e4, 4), (8 + 4 * e4, 4)], p_zlo),
                ([(8 * (1 - e8) + 4 * e4, 4)], [(a0, 4)], p_x),
                ([(a0 + (1 - e1), 1), (a0 + 2 + (1 - e1), 1)],
                 [(a0 + e1, 1), (a0 + 2 + e1, 1)], p_zhi),
                ([(a0 + 2 * (1 - e2) + e1, 1)],
                 [(a0 + 2 * e2 + e1, 1)], p_y),
            ],
        }
        cols = {"A": slice(0, S), "B": slice(S, 2 * S)}
        ssem = {"A": sendA, "B": sendB}
        rsem = {"A": rsA, "B": rsB}

        for k in range(4):
            started = []
            for pn in ("A", "B"):
                send_segs, keep_segs, partner = rs_plan[pn][k]
                so = SCR_OFF[k]
                descs = []
                for st, n in send_segs:
                    rdma = pltpu.make_async_remote_copy(
                        src_ref=accT.at[pl.ds(st * CHT, n * CHT), cols[pn]],
                        dst_ref=scr.at[so:so + n * CHT, cols[pn]],
                        send_sem=ssem[pn],
                        recv_sem=rsem[pn].at[k],
                        device_id=(partner,),
                        device_id_type=pl.DeviceIdType.MESH,
                    )
                    rdma.start()
                    descs.append(rdma)
                    so += n * CHT
                started.append((pn, descs, keep_segs))
            if k == 0:
                for b in range(B):
                    cs = slice(b * S, (b + 1) * S)
                    qT[:, cs] = DG(wq_ref[...], x_ref[b], ((0,), (1,)))
                    qrT[:, cs] = DG(wqr_ref[...], x_ref[b], ((0,), (1,)))
                    krT[:, cs] = DG(wkr_ref[...], x_ref[b], ((0,), (1,)))
            for pn, descs, keep_segs in started:
                for rdma in descs:
                    rdma.wait()
                so = SCR_OFF[k]
                for st, n in keep_segs:
                    accT[pl.ds(st * CHT, n * CHT), cols[pn]] = (
                        accT[pl.ds(st * CHT, n * CHT), cols[pn]]
                        + scr[so:so + n * CHT, cols[pn]])
                    so += n * CHT

        for b in range(B):
            cs = slice(b * S, (b + 1) * S)
            k_t = accT[pl.ds(e * CHT, DH), cs]
            v_t = accT[pl.ds(e * CHT + DH, DH), cs]
            q_t = qT[pl.ds(e * DH, DH), cs]
            qr_t = qrT[pl.ds(e * DR, DR), cs]
            s = DG(q_t, k_t, ((0,), (0,)))
            s = s + DG(qr_t, krT[:, cs], ((0,), (0,)))
            s = s * SCALE
            m = jnp.max(s, axis=1, keepdims=True)
            pr = jnp.exp(s - m)
            pr = pr / jnp.sum(pr, axis=1, keepdims=True)
            oT[pl.ds(e * CHO, CHO), cs] = DG(v_t, pr, ((1,), (1,)))

        ag_plan = {
            "A": [([((e // L) * L, L)], p)
                  for L, p in zip([1, 2, 4, 8], [p_zhi, p_y, p_zlo, p_x])],
            "B": [
                ([(e, 1)], p_y),
                ([(a4 + e1, 1), (a4 + 2 + e1, 1)], p_zhi),
                ([(a0, 4)], p_x),
                ([(4 * e4, 4), (8 + 4 * e4, 4)], p_zlo),
            ],
        }
        asem = {"A": agA, "B": agB}
        for j in range(4):
            descs = []
            for pn in ("A", "B"):
                segs, partner = ag_plan[pn][j]
                for st, n in segs:
                    rdma = pltpu.make_async_remote_copy(
                        src_ref=oT.at[pl.ds(st * CHO, n * CHO), cols[pn]],
                        dst_ref=oT.at[pl.ds(st * CHO, n * CHO), cols[pn]],
                        send_sem=ssem[pn],
                        recv_sem=asem[pn].at[j],
                        device_id=(partner,),
                        device_id_type=pl.DeviceIdType.MESH,
                    )
                    rdma.start()
                    descs.append(rdma)
            for rdma in descs:
                rdma.wait()

        for b in range(B):
            out_ref[b] = DG(oT[:, b * S:(b + 1) * S], wo_ref[...],
                            ((0,), (0,)))

    return pl.pallas_call(
        body,
        out_shape=jax.ShapeDtypeStruct((B, S, D), jnp.float32),
        in_specs=[pl.BlockSpec(memory_space=pltpu.VMEM)] * 8,
        out_specs=pl.BlockSpec(memory_space=pltpu.VMEM),
        scratch_shapes=[
            pltpu.VMEM((H * CHT, 2 * S), jnp.float32),
            pltpu.VMEM((1920, 2 * S), jnp.float32),
            pltpu.VMEM((DC, 2 * S), jnp.float32),
            pltpu.VMEM((D, 2 * S), jnp.float32),
            pltpu.VMEM((H * DR, 2 * S), jnp.float32),
            pltpu.VMEM((DR, 2 * S), jnp.float32),
            pltpu.VMEM((H * CHO, 2 * S), jnp.float32),
            pltpu.SemaphoreType.DMA,
            pltpu.SemaphoreType.DMA,
            pltpu.SemaphoreType.DMA((4,)),
            pltpu.SemaphoreType.DMA((4,)),
            pltpu.SemaphoreType.DMA((4,)),
            pltpu.SemaphoreType.DMA((4,)),
        ],
        compiler_params=pltpu.CompilerParams(collective_id=0),
    )(x, Wdkv, Wuk, Wuv, Wq, Wqr, Wkr, Wo)


# device time: 54965 ns/iter; 3.2077x vs baseline; 1.3228x over previous
import jax
import jax.numpy as jnp
from jax import lax
from jax.experimental import pallas as pl
from jax.experimental.pallas import tpu as pltpu

N_DEV = 16
B, S, D = 2, 256, 1024
DC, H, DH, DR = 64, 16, 64, 32
CHT = 2 * DH
CHO = DH
SCALE = (DH + DR) ** -0.5

BITS_A = [8, 4, 2, 1]
SCR_ROWS = [8 * CHT, 4 * CHT, 2 * CHT, 1 * CHT]
SCR_OFF = [0, 1024, 1536, 1792]
DG = lambda a, b_, dims: lax.dot_general(
    a, b_, (dims, ((), ())), preferred_element_type=jnp.float32)


def _logical(x, y, zlo, zhi):
    w = 2 * y + (x + y - 2 * x * y)
    return 4 * (zlo + 2 * zhi) + w


def kernel(x, Wdkv, Wuk, Wuv, Wq, Wqr, Wkr, Wo):
    def body(x_ref, wdkv_ref, wuk_ref, wuv_ref, wq_ref, wqr_ref, wkr_ref,
             wo_ref, out_ref, accT, scr, stg, ctT, qT, qrT, krT, oT,
             sendA, sendB, rsA, rsB, agA, agB):
        my = lax.axis_index("i")
        w = lax.rem(my, 4)
        z = my // 4
        cx = jnp.logical_or(w == 1, w == 2).astype(jnp.int32)
        cy = (w >= 2).astype(jnp.int32)
        zlo = lax.rem(z, 2)
        zhi = z // 2

        e = 8 * cx + 4 * zlo + 2 * cy + zhi
        e1 = lax.rem(e, 2)
        e2 = lax.rem(e // 2, 2)
        e4 = lax.rem(e // 4, 2)
        e8 = e // 8
        p_x = _logical(1 - cx, cy, zlo, zhi)
        p_y = _logical(cx, 1 - cy, zlo, zhi)
        p_zlo = _logical(cx, cy, 1 - zlo, zhi)
        p_zhi = _logical(cx, cy, zlo, 1 - zhi)
        a0 = 8 * e8 + 4 * e4
        a4 = (e // 4) * 4

        barrier = pltpu.get_barrier_semaphore()
        for nbr in (p_x, p_y, p_zlo, p_zhi):
            pl.semaphore_signal(barrier, inc=1, device_id=(nbr,),
                                device_id_type=pl.DeviceIdType.MESH)
        pl.semaphore_wait(barrier, 4)

        for b in range(B):
            ctT[:, b * S:(b + 1) * S] = DG(wdkv_ref[...], x_ref[b],
                                           ((0,), (1,)))
        ct = ctT[...]
        for h in range(H):
            accT[h * CHT:h * CHT + DH, :] = DG(
                wuk_ref[:, h * DH:(h + 1) * DH], ct, ((0,), (0,)))
            accT[h * CHT + DH:(h + 1) * CHT, :] = DG(
                wuv_ref[:, h * DH:(h + 1) * DH], ct, ((0,), (0,)))

        rs_plan = {
            "A": [([(((e // (2 * b_)) * (2 * b_)) + (1 - lax.rem(e // b_, 2)) * b_, b_)],
                   [(((e // (2 * b_)) * (2 * b_)) + lax.rem(e // b_, 2) * b_, b_)],
                   p) for b_, p in zip(BITS_A, [p_x, p_zlo, p_y, p_zhi])],
            "B": [
                ([(4 * (1 - e4), 4), (8 + 4 * (1 - e4), 4)],
                 [(4 * e4, 4), (8 + 4 * e4, 4)], p_zlo),
                ([(8 * (1 - e8) + 4 * e4, 4)], [(a0, 4)], p_x),
                ([(a0 + (1 - e1), 1), (a0 + 2 + (1 - e1), 1)],
                 [(a0 + e1, 1), (a0 + 2 + e1, 1)], p_zhi),
                ([(a0 + 2 * (1 - e2) + e1, 1)],
                 [(a0 + 2 * e2 + e1, 1)], p_y),
            ],
        }
        cols = {"A": slice(0, S), "B": slice(S, 2 * S)}
        ssem = {"A": sendA, "B": sendB}
        rsem = {"A": rsA, "B": rsB}

        for k in range(4):
            started = []
            for pn in ("A", "B"):
                send_segs, keep_segs, partner = rs_plan[pn][k]
                so = SCR_OFF[k]
                go = 0
                descs = []
                for st, n in send_segs:
                    stg[go:go + n * CHT, cols[pn]] = accT[
                        pl.ds(st * CHT, n * CHT), cols[pn]
                    ].astype(jnp.bfloat16)
                    rdma = pltpu.make_async_remote_copy(
                        src_ref=stg.at[go:go + n * CHT, cols[pn]],
                        dst_ref=scr.at[so:so + n * CHT, cols[pn]],
                        send_sem=ssem[pn],
                        recv_sem=rsem[pn].at[k],
                        device_id=(partner,),
                        device_id_type=pl.DeviceIdType.MESH,
                    )
                    rdma.start()
                    descs.append(rdma)
                    so += n * CHT
                    go += n * CHT
                started.append((pn, descs, keep_segs))
            if k == 0:
                for b in range(B):
                    cs = slice(b * S, (b + 1) * S)
                    qT[:, cs] = DG(wq_ref[...], x_ref[b], ((0,), (1,)))
                    qrT[:, cs] = DG(wqr_ref[...], x_ref[b], ((0,), (1,)))
                    krT[:, cs] = DG(wkr_ref[...], x_ref[b], ((0,), (1,)))
            for pn, descs, keep_segs in started:
                for rdma in descs:
                    rdma.wait()
                so = SCR_OFF[k]
                for st, n in keep_segs:
                    accT[pl.ds(st * CHT, n * CHT), cols[pn]] = (
                        accT[pl.ds(st * CHT, n * CHT), cols[pn]]
                        + scr[so:so + n * CHT, cols[pn]].astype(jnp.float32))
                    so += n * CHT

        for b in range(B):
            cs = slice(b * S, (b + 1) * S)
            k_t = accT[pl.ds(e * CHT, DH), cs]
            v_t = accT[pl.ds(e * CHT + DH, DH), cs]
            q_t = qT[pl.ds(e * DH, DH), cs]
            qr_t = qrT[pl.ds(e * DR, DR), cs]
            s = DG(q_t, k_t, ((0,), (0,)))
            s = s + DG(qr_t, krT[:, cs], ((0,), (0,)))
            s = s * SCALE
            m = jnp.max(s, axis=1, keepdims=True)
            pr = jnp.exp(s - m)
            pr = pr / jnp.sum(pr, axis=1, keepdims=True)
            oT[pl.ds(e * CHO, CHO), cs] = DG(
                v_t, pr, ((1,), (1,))).astype(jnp.bfloat16)

        ag_plan = {
            "A": [([((e // L) * L, L)], p)
                  for L, p in zip([1, 2, 4, 8], [p_zhi, p_y, p_zlo, p_x])],
            "B": [
                ([(e, 1)], p_y),
                ([(a4 + e1, 1), (a4 + 2 + e1, 1)], p_zhi),
                ([(a0, 4)], p_x),
                ([(4 * e4, 4), (8 + 4 * e4, 4)], p_zlo),
            ],
        }
        asem = {"A": agA, "B": agB}
        for j in range(4):
            descs = []
            for pn in ("A", "B"):
                segs, partner = ag_plan[pn][j]
                for st, n in segs:
                    rdma = pltpu.make_async_remote_copy(
                        src_ref=oT.at[pl.ds(st * CHO, n * CHO), cols[pn]],
                        dst_ref=oT.at[pl.ds(st * CHO, n * CHO), cols[pn]],
                        send_sem=ssem[pn],
                        recv_sem=asem[pn].at[j],
                        device_id=(partner,),
                        device_id_type=pl.DeviceIdType.MESH,
                    )
                    rdma.start()
                    descs.append(rdma)
            for rdma in descs:
                rdma.wait()

        for b in range(B):
            out_ref[b] = DG(oT[:, b * S:(b + 1) * S].astype(jnp.float32),
                            wo_ref[...], ((0,), (0,)))

    return pl.pallas_call(
        body,
        out_shape=jax.ShapeDtypeStruct((B, S, D), jnp.float32),
        in_specs=[pl.BlockSpec(memory_space=pltpu.VMEM)] * 8,
        out_specs=pl.BlockSpec(memory_space=pltpu.VMEM),
        scratch_shapes=[
            pltpu.VMEM((H * CHT, 2 * S), jnp.float32),
            pltpu.VMEM((1920, 2 * S), jnp.bfloat16),
            pltpu.VMEM((8 * CHT, 2 * S), jnp.bfloat16),
            pltpu.VMEM((DC, 2 * S), jnp.float32),
            pltpu.VMEM((D, 2 * S), jnp.float32),
            pltpu.VMEM((H * DR, 2 * S), jnp.float32),
            pltpu.VMEM((DR, 2 * S), jnp.float32),
            pltpu.VMEM((H * CHO, 2 * S), jnp.bfloat16),
            pltpu.SemaphoreType.DMA,
            pltpu.SemaphoreType.DMA,
            pltpu.SemaphoreType.DMA((4,)),
            pltpu.SemaphoreType.DMA((4,)),
            pltpu.SemaphoreType.DMA((4,)),
            pltpu.SemaphoreType.DMA((4,)),
        ],
        compiler_params=pltpu.CompilerParams(collective_id=0),
    )(x, Wdkv, Wuk, Wuv, Wq, Wqr, Wkr, Wo)


# device time: 54694 ns/iter; 3.2236x vs baseline; 1.0050x over previous
import jax
import jax.numpy as jnp
from jax import lax
from jax.experimental import pallas as pl
from jax.experimental.pallas import tpu as pltpu

N_DEV = 16
B, S, D = 2, 256, 1024
DC, H, DH, DR = 64, 16, 64, 32
CHT = 2 * DH
CHO = DH
SCALE = (DH + DR) ** -0.5

BITS_A = [8, 4, 2, 1]
SCR_ROWS = [8 * CHT, 4 * CHT, 2 * CHT, 1 * CHT]
SCR_OFF = [0, 1024, 1536, 1792]
DG = lambda a, b_, dims: lax.dot_general(
    a, b_, (dims, ((), ())), preferred_element_type=jnp.float32)


def _logical(x, y, zlo, zhi):
    w = 2 * y + (x + y - 2 * x * y)
    return 4 * (zlo + 2 * zhi) + w


def kernel(x, Wdkv, Wuk, Wuv, Wq, Wqr, Wkr, Wo):
    def body(x_ref, wdkv_ref, wuk_ref, wuv_ref, wq_ref, wqr_ref, wkr_ref,
             wo_ref, out_ref, accT, scr, stg, ctT, qT, qrT, krT, oT,
             sendA, sendB, rsA, rsB, agA, agB):
        my = lax.axis_index("i")
        w = lax.rem(my, 4)
        z = my // 4
        cx = jnp.logical_or(w == 1, w == 2).astype(jnp.int32)
        cy = (w >= 2).astype(jnp.int32)
        zlo = lax.rem(z, 2)
        zhi = z // 2

        e = 8 * cx + 4 * zlo + 2 * cy + zhi
        e1 = lax.rem(e, 2)
        e2 = lax.rem(e // 2, 2)
        e4 = lax.rem(e // 4, 2)
        e8 = e // 8
        p_x = _logical(1 - cx, cy, zlo, zhi)
        p_y = _logical(cx, 1 - cy, zlo, zhi)
        p_zlo = _logical(cx, cy, 1 - zlo, zhi)
        p_zhi = _logical(cx, cy, zlo, 1 - zhi)
        a0 = 8 * e8 + 4 * e4
        a4 = (e // 4) * 4

        barrier = pltpu.get_barrier_semaphore()
        for nbr in (p_x, p_y, p_zlo, p_zhi):
            pl.semaphore_signal(barrier, inc=1, device_id=(nbr,),
                                device_id_type=pl.DeviceIdType.MESH)
        pl.semaphore_wait(barrier, 4)

        for b in range(B):
            ctT[:, b * S:(b + 1) * S] = DG(wdkv_ref[...], x_ref[b],
                                           ((0,), (1,)))
        ct = ctT[...]
        for h in range(H):
            accT[h * CHT:h * CHT + DH, :] = DG(
                wuk_ref[:, h * DH:(h + 1) * DH], ct, ((0,), (0,)))
            accT[h * CHT + DH:(h + 1) * CHT, :] = DG(
                wuv_ref[:, h * DH:(h + 1) * DH], ct, ((0,), (0,)))

        rs_plan = {
            "A": [([(((e // (2 * b_)) * (2 * b_)) + (1 - lax.rem(e // b_, 2)) * b_, b_)],
                   [(((e // (2 * b_)) * (2 * b_)) + lax.rem(e // b_, 2) * b_, b_)],
                   p) for b_, p in zip(BITS_A, [p_x, p_zlo, p_y, p_zhi])],
            "B": [
                ([(4 * (1 - e4), 4), (8 + 4 * (1 - e4), 4)],
                 [(4 * e4, 4), (8 + 4 * e4, 4)], p_zlo),
                ([(8 * (1 - e8) + 4 * e4, 4)], [(a0, 4)], p_x),
                ([(a0 + (1 - e1), 1), (a0 + 2 + (1 - e1), 1)],
                 [(a0 + e1, 1), (a0 + 2 + e1, 1)], p_zhi),
                ([(a0 + 2 * (1 - e2) + e1, 1)],
                 [(a0 + 2 * e2 + e1, 1)], p_y),
            ],
        }
        cols = {"A": slice(0, S), "B": slice(S, 2 * S)}
        ssem = {"A": sendA, "B": sendB}
        rsem = {"A": rsA, "B": rsB}

        for k in range(4):
            started = []
            for pn in ("A", "B"):
                send_segs, keep_segs, partner = rs_plan[pn][k]
                so = SCR_OFF[k]
                go = 0
                descs = []
                for st, n in send_segs:
                    stg[go:go + n * CHT, cols[pn]] = accT[
                        pl.ds(st * CHT, n * CHT), cols[pn]
                    ].astype(jnp.bfloat16)
                    rdma = pltpu.make_async_remote_copy(
                        src_ref=stg.at[go:go + n * CHT, cols[pn]],
                        dst_ref=scr.at[so:so + n * CHT, cols[pn]],
                        send_sem=ssem[pn],
                        recv_sem=rsem[pn].at[k],
                        device_id=(partner,),
                        device_id_type=pl.DeviceIdType.MESH,
                    )
                    rdma.start()
                    descs.append(rdma)
                    so += n * CHT
                    go += n * CHT
                started.append((pn, descs, keep_segs))
            if k == 0:
                qT[:, 0:S] = DG(wq_ref[...], x_ref[0], ((0,), (1,)))
            elif k == 1:
                qT[:, S:2 * S] = DG(wq_ref[...], x_ref[1], ((0,), (1,)))
            elif k == 2:
                for b in range(B):
                    cs = slice(b * S, (b + 1) * S)
                    qrT[:, cs] = DG(wqr_ref[...], x_ref[b], ((0,), (1,)))
                    krT[:, cs] = DG(wkr_ref[...], x_ref[b], ((0,), (1,)))
            for pn, descs, keep_segs in started:
                for rdma in descs:
                    rdma.wait()
                so = SCR_OFF[k]
                for st, n in keep_segs:
                    accT[pl.ds(st * CHT, n * CHT), cols[pn]] = (
                        accT[pl.ds(st * CHT, n * CHT), cols[pn]]
                        + scr[so:so + n * CHT, cols[pn]].astype(jnp.float32))
                    so += n * CHT

        for b in range(B):
            cs = slice(b * S, (b + 1) * S)
            k_t = accT[pl.ds(e * CHT, DH), cs]
            v_t = accT[pl.ds(e * CHT + DH, DH), cs]
            q_t = qT[pl.ds(e * DH, DH), cs]
            qr_t = qrT[pl.ds(e * DR, DR), cs]
            s = DG(q_t, k_t, ((0,), (0,)))
            s = s + DG(qr_t, krT[:, cs], ((0,), (0,)))
            s = s * SCALE
            m = jnp.max(s, axis=1, keepdims=True)
            pr = jnp.exp(s - m)
            pr = pr / jnp.sum(pr, axis=1, keepdims=True)
            oT[pl.ds(e * CHO, CHO), cs] = DG(
                v_t, pr, ((1,), (1,))).astype(jnp.bfloat16)

        descs = []
        for m in range(1, N_DEV):
            mx, mzlo, my_, mzhi = (m >> 3) & 1, (m >> 2) & 1, (m >> 1) & 1, m & 1
            peer = _logical(cx + mx - 2 * cx * mx, cy + my_ - 2 * cy * my_,
                            zlo + mzlo - 2 * zlo * mzlo,
                            zhi + mzhi - 2 * zhi * mzhi)
            rdma = pltpu.make_async_remote_copy(
                src_ref=oT.at[pl.ds(e * CHO, CHO), :],
                dst_ref=oT.at[pl.ds(e * CHO, CHO), :],
                send_sem=sendA,
                recv_sem=agA.at[0],
                device_id=(peer,),
                device_id_type=pl.DeviceIdType.MESH,
            )
            rdma.start()
            descs.append(rdma)
        for rdma in descs:
            rdma.wait_recv()

        for b in range(B):
            out_ref[b] = DG(oT[:, b * S:(b + 1) * S].astype(jnp.float32),
                            wo_ref[...], ((0,), (0,)))

        for rdma in descs:
            rdma.wait_send()

    return pl.pallas_call(
        body,
        out_shape=jax.ShapeDtypeStruct((B, S, D), jnp.float32),
        in_specs=[pl.BlockSpec(memory_space=pltpu.VMEM)] * 8,
        out_specs=pl.BlockSpec(memory_space=pltpu.VMEM),
        scratch_shapes=[
            pltpu.VMEM((H * CHT, 2 * S), jnp.float32),
            pltpu.VMEM((1920, 2 * S), jnp.bfloat16),
            pltpu.VMEM((8 * CHT, 2 * S), jnp.bfloat16),
            pltpu.VMEM((DC, 2 * S), jnp.float32),
            pltpu.VMEM((D, 2 * S), jnp.float32),
            pltpu.VMEM((H * DR, 2 * S), jnp.float32),
            pltpu.VMEM((DR, 2 * S), jnp.float32),
            pltpu.VMEM((H * CHO, 2 * S), jnp.bfloat16),
            pltpu.SemaphoreType.DMA,
            pltpu.SemaphoreType.DMA,
            pltpu.SemaphoreType.DMA((4,)),
            pltpu.SemaphoreType.DMA((4,)),
            pltpu.SemaphoreType.DMA((4,)),
            pltpu.SemaphoreType.DMA((4,)),
        ],
        compiler_params=pltpu.CompilerParams(collective_id=0),
    )(x, Wdkv, Wuk, Wuv, Wq, Wqr, Wkr, Wo)
